# baseline (speedup 1.0000x reference)
import functools

import jax
import jax.numpy as jnp
from jax import lax
from jax.experimental import pallas as pl
from jax.experimental.pallas import tpu as pltpu
from jax.experimental.pallas import tpu_sc as plsc

_NUM_CORES = 2
_NUM_SUBCORES = 16
_NUM_WORKERS = _NUM_CORES * _NUM_SUBCORES
_CHUNK = 512


def _make_gather(n_total, depth):
    per_w = n_total // _NUM_WORKERS
    n_chunks = per_w // _CHUNK
    assert per_w % _CHUNK == 0

    mesh = plsc.VectorSubcoreMesh(
        core_axis_name="c",
        subcore_axis_name="s",
        num_cores=_NUM_CORES,
        num_subcores=_NUM_SUBCORES,
    )

    @functools.partial(
        pl.kernel,
        out_type=jax.ShapeDtypeStruct((n_total, depth), jnp.float32),
        mesh=mesh,
        scratch_types=[
            pltpu.VMEM((_CHUNK,), jnp.int32),
            pltpu.VMEM((_CHUNK, depth), jnp.float32),
            pltpu.SemaphoreType.DMA,
        ],
        compiler_params=pltpu.CompilerParams(use_tc_tiling_on_sc=False),
    )
    def gather_kernel(idx_hbm, table_hbm, out_hbm, idx_v, rows_v, sem):
        wid = lax.axis_index("s") * _NUM_CORES + lax.axis_index("c")
        base = wid * per_w

        def body(i, carry):
            off = base + i * _CHUNK
            pltpu.sync_copy(idx_hbm.at[pl.ds(off, _CHUNK)], idx_v)
            pltpu.async_copy(table_hbm.at[idx_v], rows_v, sem).wait()
            pltpu.sync_copy(rows_v, out_hbm.at[pl.ds(off, _CHUNK)])
            return carry

        lax.fori_loop(0, n_chunks, body, 0)

    return gather_kernel


def kernel(indices, embedding):
    b, f = indices.shape
    _, d = embedding.shape
    n = b * f
    flat_idx = indices.reshape(n).astype(jnp.int32)
    out = _make_gather(n, d)(flat_idx, embedding)
    return out.reshape(b, f, d)


# speedup vs baseline: 1.0305x; 1.0305x over previous
import functools

import jax
import jax.numpy as jnp
from jax import lax
from jax.experimental import pallas as pl
from jax.experimental.pallas import tpu as pltpu
from jax.experimental.pallas import tpu_sc as plsc

_NUM_CORES = 2
_NUM_SUBCORES = 16
_NUM_WORKERS = _NUM_CORES * _NUM_SUBCORES
_CHUNK = 832


def _make_gather(n_total, depth):
    per_w = n_total // _NUM_WORKERS
    n_chunks = per_w // _CHUNK
    assert per_w % _CHUNK == 0 and n_chunks % 2 == 0

    mesh = plsc.VectorSubcoreMesh(
        core_axis_name="c",
        subcore_axis_name="s",
        num_cores=_NUM_CORES,
        num_subcores=_NUM_SUBCORES,
    )

    @functools.partial(
        pl.kernel,
        out_type=jax.ShapeDtypeStruct((n_total, depth), jnp.float32),
        mesh=mesh,
        scratch_types=[
            pltpu.VMEM((per_w,), jnp.int32),
            pltpu.VMEM((_CHUNK, depth), jnp.float32),
            pltpu.VMEM((_CHUNK, depth), jnp.float32),
            pltpu.SemaphoreType.DMA,
            pltpu.SemaphoreType.DMA,
            pltpu.SemaphoreType.DMA,
            pltpu.SemaphoreType.DMA,
        ],
        compiler_params=pltpu.CompilerParams(use_tc_tiling_on_sc=False),
    )
    def gather_kernel(idx_hbm, table_hbm, out_hbm, idx_v, rows0, rows1,
                      gsem0, gsem1, wsem0, wsem1):
        wid = lax.axis_index("s") * _NUM_CORES + lax.axis_index("c")
        base = wid * per_w
        rows = (rows0, rows1)
        gsems = (gsem0, gsem1)
        wsems = (wsem0, wsem1)

        pltpu.sync_copy(idx_hbm.at[pl.ds(base, per_w)], idx_v)

        def start_gather(i, b):
            pltpu.async_copy(
                table_hbm.at[idx_v.at[pl.ds(i * _CHUNK, _CHUNK)]],
                rows[b], gsems[b])

        def wait_gather(b):
            pltpu.make_async_copy(
                table_hbm.at[idx_v.at[pl.ds(0, _CHUNK)]],
                rows[b], gsems[b]).wait()

        def start_write(i, b):
            pltpu.async_copy(
                rows[b], out_hbm.at[pl.ds(base + i * _CHUNK, _CHUNK)],
                wsems[b])

        def wait_write(b):
            pltpu.make_async_copy(
                rows[b], out_hbm.at[pl.ds(base, _CHUNK)], wsems[b]).wait()

        start_gather(0, 0)

        def body(j, carry):
            for b in range(2):
                i = 2 * j + b
                nb = 1 - b

                @pl.when(i >= 1)
                def _():
                    wait_write(nb)

                @pl.when(i + 1 < n_chunks)
                def _():
                    start_gather(i + 1, nb)

                wait_gather(b)
                start_write(i, b)
            return carry

        lax.fori_loop(0, n_chunks // 2, body, 0)
        wait_write(1)

    return gather_kernel


def kernel(indices, embedding):
    b, f = indices.shape
    _, d = embedding.shape
    n = b * f
    flat_idx = indices.reshape(n).astype(jnp.int32)
    out = _make_gather(n, d)(flat_idx, embedding)
    return out.reshape(b, f, d)
